# Initial kernel scaffold; baseline (speedup 1.0000x reference)
import jax
import jax.numpy as jnp
from jax.experimental import pallas as pl


def kernel(x, indices, W, b, qw_q, qw_k, qw_v, sc_q, sc_k, sc_v):
    raise NotImplementedError("write your pallas kernel here")


# speedup vs baseline: 1.0733x; 1.0733x over previous
import jax
import jax.numpy as jnp
from jax.experimental import pallas as pl
from jax.experimental.pallas import tpu as pltpu

D_MODEL = 2048
MAX_DELTAS = 4
PACK = 8
TOKENS = 2048
OUT = 3072
PCOLS = D_MODEL // PACK

TB = 256
OB = 256


def _body(idx_ref, x_ref, xp_ref, w_ref, b_ref, qw_ref, sc_ref, o_ref, wd_ref):
    tb = pl.program_id(1)

    @pl.when(tb == 0)
    def _dequant():
        qw = qw_ref[...]
        for p in range(PACK):
            nib = ((qw >> (4 * p)) & 0xF).astype(jnp.float32)
            wd_ref[:, :, p * PCOLS:(p + 1) * PCOLS] = nib

    x = x_ref[...]
    xp = xp_ref[...]
    idx = idx_ref[...]
    dn = (((1,), (1,)), ((), ()))
    acc = jax.lax.dot_general(x, w_ref[...], dn,
                              preferred_element_type=jnp.float32)
    rs = jnp.sum(x, axis=1, keepdims=True)
    for d in range(MAX_DELTAS):
        m = (idx == d).astype(jnp.float32)
        y = jax.lax.dot_general(xp * m, wd_ref[d], dn,
                                preferred_element_type=jnp.float32)
        acc = acc + sc_ref[d] * (y - 8.0 * (rs * m))
    o_ref[...] = acc + b_ref[...]


def kernel(x, indices, W, b, qw_q, qw_k, qw_v, sc_q, sc_k, sc_v):
    qw = jnp.concatenate([qw_q, qw_k, qw_v], axis=1)
    sc = jnp.concatenate([sc_q, sc_k, sc_v], axis=1)
    sc = sc.reshape(MAX_DELTAS, 1, OUT)
    b2 = b.reshape(1, OUT)
    idx2 = indices.reshape(TOKENS, 1)
    xp = x.reshape(TOKENS, PCOLS, PACK).transpose(0, 2, 1).reshape(TOKENS, D_MODEL)

    grid = (OUT // OB, TOKENS // TB)
    out = pl.pallas_call(
        _body,
        grid=grid,
        in_specs=[
            pl.BlockSpec((TB, 1), lambda ob, tb: (tb, 0)),
            pl.BlockSpec((TB, D_MODEL), lambda ob, tb: (tb, 0)),
            pl.BlockSpec((TB, D_MODEL), lambda ob, tb: (tb, 0)),
            pl.BlockSpec((OB, D_MODEL), lambda ob, tb: (ob, 0)),
            pl.BlockSpec((1, OB), lambda ob, tb: (0, ob)),
            pl.BlockSpec((MAX_DELTAS, OB, PCOLS), lambda ob, tb: (0, ob, 0)),
            pl.BlockSpec((MAX_DELTAS, 1, OB), lambda ob, tb: (0, 0, ob)),
        ],
        out_specs=pl.BlockSpec((TB, OB), lambda ob, tb: (tb, ob)),
        out_shape=jax.ShapeDtypeStruct((TOKENS, OUT), jnp.float32),
        scratch_shapes=[pltpu.VMEM((MAX_DELTAS, OB, D_MODEL), jnp.float32)],
    )(idx2, x, xp, W, b2, qw, sc)
    return out


# speedup vs baseline: 1.1746x; 1.0944x over previous
import jax
import jax.numpy as jnp
from jax.experimental import pallas as pl
from jax.experimental.pallas import tpu as pltpu

D_MODEL = 2048
MAX_DELTAS = 4
PACK = 8
TOKENS = 2048
OUT = 3072
PCOLS = D_MODEL // PACK

TB = 256
OB = 256


def _body(idx_ref, x_ref, xp_ref, w_ref, b_ref, qw_ref, sc_ref, o_ref, wd_ref):
    tb = pl.program_id(1)

    @pl.when(tb == 0)
    def _dequant():
        qw = qw_ref[...]
        for p in range(PACK):
            nib = ((qw >> (4 * p)) & 0xF).astype(jnp.bfloat16)
            wd_ref[:, :, p * PCOLS:(p + 1) * PCOLS] = nib

    x = x_ref[...]
    xp = xp_ref[...]
    idx = idx_ref[...]
    dn = (((1,), (1,)), ((), ()))
    acc = jax.lax.dot_general(x, w_ref[...], dn,
                              preferred_element_type=jnp.float32)
    rs = jnp.sum(x.astype(jnp.float32), axis=1, keepdims=True)
    for d in range(MAX_DELTAS):
        m = (idx == d).astype(jnp.float32)
        y = jax.lax.dot_general(xp * m.astype(jnp.bfloat16), wd_ref[d], dn,
                                preferred_element_type=jnp.float32)
        acc = acc + sc_ref[d] * (y - 8.0 * (rs * m))
    o_ref[...] = acc + b_ref[...]


def kernel(x, indices, W, b, qw_q, qw_k, qw_v, sc_q, sc_k, sc_v):
    qw = jnp.concatenate([qw_q, qw_k, qw_v], axis=1)
    sc = jnp.concatenate([sc_q, sc_k, sc_v], axis=1)
    sc = sc.reshape(MAX_DELTAS, 1, OUT)
    b2 = b.reshape(1, OUT)
    idx2 = indices.reshape(TOKENS, 1)
    xp = x.reshape(TOKENS, PCOLS, PACK).transpose(0, 2, 1).reshape(TOKENS, D_MODEL)
    x = x.astype(jnp.bfloat16)
    xp = xp.astype(jnp.bfloat16)
    W = W.astype(jnp.bfloat16)

    grid = (OUT // OB, TOKENS // TB)
    out = pl.pallas_call(
        _body,
        grid=grid,
        in_specs=[
            pl.BlockSpec((TB, 1), lambda ob, tb: (tb, 0)),
            pl.BlockSpec((TB, D_MODEL), lambda ob, tb: (tb, 0)),
            pl.BlockSpec((TB, D_MODEL), lambda ob, tb: (tb, 0)),
            pl.BlockSpec((OB, D_MODEL), lambda ob, tb: (ob, 0)),
            pl.BlockSpec((1, OB), lambda ob, tb: (0, ob)),
            pl.BlockSpec((MAX_DELTAS, OB, PCOLS), lambda ob, tb: (0, ob, 0)),
            pl.BlockSpec((MAX_DELTAS, 1, OB), lambda ob, tb: (0, 0, ob)),
        ],
        out_specs=pl.BlockSpec((TB, OB), lambda ob, tb: (tb, ob)),
        out_shape=jax.ShapeDtypeStruct((TOKENS, OUT), jnp.float32),
        scratch_shapes=[pltpu.VMEM((MAX_DELTAS, OB, D_MODEL), jnp.bfloat16)],
    )(idx2, x, xp, W, b2, qw, sc)
    return out
